# Initial kernel scaffold; baseline (speedup 1.0000x reference)
import jax
import jax.numpy as jnp
from jax.experimental import pallas as pl


def kernel(x, edge_index, W1, W_mu, W_logvar, eps):
    raise NotImplementedError("write your pallas kernel here")


# speedup vs baseline: 3.7539x; 3.7539x over previous
import functools

import jax
import jax.numpy as jnp
from jax import lax
from jax.experimental import pallas as pl
from jax.experimental.pallas import tpu as pltpu
from jax.experimental.pallas import tpu_sc as plsc

NN = 10000
EE = 160000
DD = 256
HH = 256
ZD = 64
FH = 128

NC, NS, LL = 2, 16, 16
CHUNK = 128
NPAD = 10240
EPAD = 163840
RPT = NPAD // NS
ECH = EPAD // CHUNK
DUMMY = NN

_mesh = plsc.VectorSubcoreMesh(
    core_axis_name="c", subcore_axis_name="s", num_cores=NC, num_subcores=NS
)

DCH = ECH // NS


@functools.partial(
    pl.kernel,
    out_type=jax.ShapeDtypeStruct((2, NPAD), jnp.float32),
    mesh=_mesh,
    scratch_types=[
        pltpu.VMEM_SHARED((NPAD,), jnp.float32),
        pltpu.VMEM((DCH, CHUNK), jnp.int32),
        pltpu.VMEM((CHUNK,), jnp.float32),
    ],
)
def _deg_sc(src_hbm, dst_hbm, zeros1_hbm, ones_hbm, out_hbm, hist, idx, ones_v):
    c = lax.axis_index("c")
    s = lax.axis_index("s")
    pltpu.sync_copy(zeros1_hbm.at[pl.ds(s * RPT, RPT)], hist.at[pl.ds(s * RPT, RPT)])
    pltpu.sync_copy(ones_hbm, ones_v)

    def run(idx_hbm):
        pltpu.sync_copy(idx_hbm.at[pl.ds(s * DCH, DCH)], idx)
        plsc.subcore_barrier()

        def body(j, carry):
            pltpu.sync_copy(ones_v, hist.at[idx.at[j]], add=True)
            return carry

        lax.fori_loop(0, DCH, body, 0)

    @pl.when(c == 0)
    def _():
        run(src_hbm)

    @pl.when(c == 1)
    def _():
        run(dst_hbm)

    plsc.subcore_barrier()

    @pl.when(s == 0)
    def _():
        pltpu.sync_copy(hist, out_hbm.at[c])


S1CH = ECH // NS


@functools.partial(
    pl.kernel,
    out_type=(
        jax.ShapeDtypeStruct((NPAD, FH), jnp.float32),
        jax.ShapeDtypeStruct((NPAD, FH), jnp.float32),
    ),
    mesh=_mesh,
    scratch_types=[
        pltpu.VMEM_SHARED((NPAD, FH), jnp.float32),
        pltpu.VMEM((S1CH, CHUNK), jnp.int32),
        pltpu.VMEM((S1CH, CHUNK), jnp.int32),
        pltpu.VMEM((CHUNK, FH), jnp.float32),
        pltpu.SemaphoreType.DMA,
    ],
)
def _seg1_sc(hwa, hwb, srcm, dstm, zeros2_hbm, outa, outb, agg, idxs, idxd, rows, sem):
    c = lax.axis_index("c")
    s = lax.axis_index("s")
    pltpu.sync_copy(
        zeros2_hbm.at[pl.ds(s * RPT, RPT)], agg.at[pl.ds(s * RPT, RPT)]
    )
    pltpu.sync_copy(srcm.at[pl.ds(s * S1CH, S1CH)], idxs)
    pltpu.sync_copy(dstm.at[pl.ds(s * S1CH, S1CH)], idxd)
    plsc.subcore_barrier()

    def run(tab):
        def body(j, carry):
            pltpu.async_copy(tab.at[idxs.at[j]], rows, sem).wait()
            pltpu.sync_copy(rows, agg.at[idxd.at[j]], add=True)
            return carry

        lax.fori_loop(0, S1CH, body, 0)

    @pl.when(c == 0)
    def _():
        run(hwa)

    @pl.when(c == 1)
    def _():
        run(hwb)

    plsc.subcore_barrier()

    def wout(o):
        pltpu.sync_copy(agg.at[pl.ds(s * RPT, RPT)], o.at[pl.ds(s * RPT, RPT)])

    @pl.when(c == 0)
    def _():
        wout(outa)

    @pl.when(c == 1)
    def _():
        wout(outb)


S2CH = ECH // (NC * NS)


@functools.partial(
    pl.kernel,
    out_type=(
        jax.ShapeDtypeStruct((NPAD, 2 * ZD), jnp.float32),
        jax.ShapeDtypeStruct((NPAD, 2 * ZD), jnp.float32),
    ),
    mesh=_mesh,
    scratch_types=[
        pltpu.VMEM_SHARED((NPAD, FH), jnp.float32),
        pltpu.VMEM((S2CH, CHUNK), jnp.int32),
        pltpu.VMEM((S2CH, CHUNK), jnp.int32),
        pltpu.VMEM((CHUNK, FH), jnp.float32),
        pltpu.SemaphoreType.DMA,
    ],
)
def _seg2_sc(tab, srcm, dstm, zeros2_hbm, out0, out1, agg, idxs, idxd, rows, sem):
    c = lax.axis_index("c")
    s = lax.axis_index("s")
    pltpu.sync_copy(
        zeros2_hbm.at[pl.ds(s * RPT, RPT)], agg.at[pl.ds(s * RPT, RPT)]
    )
    base = (c * NS + s) * S2CH
    pltpu.sync_copy(srcm.at[pl.ds(base, S2CH)], idxs)
    pltpu.sync_copy(dstm.at[pl.ds(base, S2CH)], idxd)
    plsc.subcore_barrier()

    def body(j, carry):
        pltpu.async_copy(tab.at[idxs.at[j]], rows, sem).wait()
        pltpu.sync_copy(rows, agg.at[idxd.at[j]], add=True)
        return carry

    lax.fori_loop(0, S2CH, body, 0)
    plsc.subcore_barrier()

    def wout(o):
        pltpu.sync_copy(agg.at[pl.ds(s * RPT, RPT)], o.at[pl.ds(s * RPT, RPT)])

    @pl.when(c == 0)
    def _():
        wout(out0)

    @pl.when(c == 1)
    def _():
        wout(out1)


RB1 = 1024


def _mm1_body(x_ref, w_ref, deg_ref, outa_ref, outb_ref):
    ns = lax.rsqrt(jnp.maximum(deg_ref[...], 1.0))
    y = jnp.dot(x_ref[...], w_ref[...], preferred_element_type=jnp.float32)
    y = y * ns[:, None]
    outa_ref[...] = y[:, :FH]
    outb_ref[...] = y[:, FH:]


_mm1 = pl.pallas_call(
    _mm1_body,
    grid=(NPAD // RB1,),
    in_specs=[
        pl.BlockSpec((RB1, DD), lambda i: (i, 0)),
        pl.BlockSpec((DD, HH), lambda i: (0, 0)),
        pl.BlockSpec((RB1,), lambda i: (i,)),
    ],
    out_specs=[
        pl.BlockSpec((RB1, FH), lambda i: (i, 0)),
        pl.BlockSpec((RB1, FH), lambda i: (i, 0)),
    ],
    out_shape=[
        jax.ShapeDtypeStruct((NPAD, FH), jnp.float32),
        jax.ShapeDtypeStruct((NPAD, FH), jnp.float32),
    ],
)


def _mm2_body(a_ref, b_ref, degd_ref, degs_ref, w_ref, out_ref):
    nd = lax.rsqrt(jnp.maximum(degd_ref[...], 1.0))
    ns = lax.rsqrt(jnp.maximum(degs_ref[...], 1.0))
    h = jnp.concatenate([a_ref[...], b_ref[...]], axis=1) * nd[:, None]
    h = jnp.maximum(h, 0.0)
    y = jnp.dot(h, w_ref[...], preferred_element_type=jnp.float32)
    out_ref[...] = y * ns[:, None]


_mm2 = pl.pallas_call(
    _mm2_body,
    grid=(NPAD // RB1,),
    in_specs=[
        pl.BlockSpec((RB1, FH), lambda i: (i, 0)),
        pl.BlockSpec((RB1, FH), lambda i: (i, 0)),
        pl.BlockSpec((RB1,), lambda i: (i,)),
        pl.BlockSpec((RB1,), lambda i: (i,)),
        pl.BlockSpec((HH, 2 * ZD), lambda i: (0, 0)),
    ],
    out_specs=pl.BlockSpec((RB1, 2 * ZD), lambda i: (i, 0)),
    out_shape=jax.ShapeDtypeStruct((NPAD, 2 * ZD), jnp.float32),
)


BR = 512
BC = 1024
GI = -(-NN // BR)
GJ = -(-NN // BC)


def _dec_body(p0i, p1i, di, ei, p0j, p1j, dj, ej, out_ref):
    def mkz(p0, p1, dg, ep):
        nd = lax.rsqrt(jnp.maximum(dg[...], 1.0))[:, None]
        sm = p0[...] + p1[...]
        mu = sm[:, :ZD] * nd
        log_var = sm[:, ZD:] * nd
        return mu + jnp.exp(0.5 * log_var) * ep[...]

    zi = mkz(p0i, p1i, di, ei)
    zj = mkz(p0j, p1j, dj, ej)
    prod = lax.dot_general(
        zi, zj, (((1,), (1,)), ((), ())), preferred_element_type=jnp.float32
    )
    out_ref[...] = jax.nn.sigmoid(prod)


_dec = pl.pallas_call(
    _dec_body,
    grid=(GI, GJ),
    in_specs=[
        pl.BlockSpec((BR, 2 * ZD), lambda i, j: (i, 0)),
        pl.BlockSpec((BR, 2 * ZD), lambda i, j: (i, 0)),
        pl.BlockSpec((BR,), lambda i, j: (i,)),
        pl.BlockSpec((BR, ZD), lambda i, j: (i, 0)),
        pl.BlockSpec((BC, 2 * ZD), lambda i, j: (j, 0)),
        pl.BlockSpec((BC, 2 * ZD), lambda i, j: (j, 0)),
        pl.BlockSpec((BC,), lambda i, j: (j,)),
        pl.BlockSpec((BC, ZD), lambda i, j: (j, 0)),
    ],
    out_specs=pl.BlockSpec((BR, BC), lambda i, j: (i, j)),
    out_shape=jax.ShapeDtypeStruct((NN, NN), jnp.float32),
)


def kernel(x, edge_index, W1, W_mu, W_logvar, eps):
    src = edge_index[0]
    dst = edge_index[1]
    pad = jnp.full((EPAD - EE,), DUMMY, jnp.int32)
    srcm = jnp.concatenate([src, pad]).reshape(ECH, CHUNK)
    dstm = jnp.concatenate([dst, pad]).reshape(ECH, CHUNK)
    x_pad = jnp.pad(x, ((0, NPAD - NN), (0, 0)))
    eps_pad = jnp.pad(eps, ((0, NPAD - NN), (0, 0)))
    zeros1 = jnp.zeros((NPAD,), jnp.float32)
    zeros2 = jnp.zeros((NPAD, FH), jnp.float32)
    ones1 = jnp.ones((CHUNK,), jnp.float32)
    w_cat = jnp.concatenate([W_mu, W_logvar], axis=1)

    deg = _deg_sc(srcm, dstm, zeros1, ones1)
    deg_src = deg[0]
    deg_dst = deg[1]
    hwa, hwb = _mm1(x_pad, W1, deg_src)
    agga, aggb = _seg1_sc(hwa, hwb, srcm, dstm, zeros2)
    mm2o = _mm2(agga, aggb, deg_dst, deg_src, w_cat)
    p0, p1 = _seg2_sc(mm2o, srcm, dstm, zeros2)
    return _dec(p0, p1, deg_dst, eps_pad, p0, p1, deg_dst, eps_pad)


# speedup vs baseline: 4.0495x; 1.0787x over previous
import functools

import jax
import jax.numpy as jnp
from jax import lax
from jax.experimental import pallas as pl
from jax.experimental.pallas import tpu as pltpu
from jax.experimental.pallas import tpu_sc as plsc

NN = 10000
EE = 160000
DD = 256
HH = 256
ZD = 64
FH = 128

NC, NS, LL = 2, 16, 16
CHUNK = 128
NPAD = 10240
EPAD = 163840
RPT = NPAD // NS
ECH = EPAD // CHUNK
DUMMY = NN

_mesh = plsc.VectorSubcoreMesh(
    core_axis_name="c", subcore_axis_name="s", num_cores=NC, num_subcores=NS
)

DCH = ECH // NS


@functools.partial(
    pl.kernel,
    out_type=jax.ShapeDtypeStruct((2, NPAD), jnp.float32),
    mesh=_mesh,
    scratch_types=[
        pltpu.VMEM_SHARED((NPAD,), jnp.float32),
        pltpu.VMEM((DCH, CHUNK), jnp.int32),
        pltpu.VMEM((CHUNK,), jnp.float32),
    ],
)
def _deg_sc(src_hbm, dst_hbm, zeros1_hbm, ones_hbm, out_hbm, hist, idx, ones_v):
    c = lax.axis_index("c")
    s = lax.axis_index("s")
    pltpu.sync_copy(zeros1_hbm.at[pl.ds(s * RPT, RPT)], hist.at[pl.ds(s * RPT, RPT)])
    pltpu.sync_copy(ones_hbm, ones_v)

    def run(idx_hbm):
        pltpu.sync_copy(idx_hbm.at[pl.ds(s * DCH, DCH)], idx)
        plsc.subcore_barrier()

        def body(j, carry):
            pltpu.sync_copy(ones_v, hist.at[idx.at[j]], add=True)
            return carry

        lax.fori_loop(0, DCH, body, 0)

    @pl.when(c == 0)
    def _():
        run(src_hbm)

    @pl.when(c == 1)
    def _():
        run(dst_hbm)

    plsc.subcore_barrier()

    @pl.when(s == 0)
    def _():
        pltpu.sync_copy(hist, out_hbm.at[c])


S1CH = ECH // NS


IBCH = 8


def _seg_loop(tab, agg, srcm, dstm, base, nch, idxs, idxd, rows0, rows1, sem):
    def stage(st, carry):
        pltpu.sync_copy(srcm.at[pl.ds(base + st * IBCH, IBCH)], idxs)
        pltpu.sync_copy(dstm.at[pl.ds(base + st * IBCH, IBCH)], idxd)
        pltpu.async_copy(tab.at[idxs.at[0]], rows0, sem)

        def body(k, c2):
            j0 = 2 * k
            pltpu.async_copy(tab.at[idxs.at[j0 + 1]], rows1, sem)
            pltpu.make_async_copy(tab.at[idxs.at[j0]], rows0, sem).wait()
            pltpu.sync_copy(rows0, agg.at[idxd.at[j0]], add=True)

            @pl.when(k < IBCH // 2 - 1)
            def _():
                pltpu.async_copy(tab.at[idxs.at[j0 + 2]], rows0, sem)

            pltpu.make_async_copy(tab.at[idxs.at[j0 + 1]], rows1, sem).wait()
            pltpu.sync_copy(rows1, agg.at[idxd.at[j0 + 1]], add=True)
            return c2

        lax.fori_loop(0, IBCH // 2, body, 0)
        return carry

    lax.fori_loop(0, nch // IBCH, stage, 0)


@functools.partial(
    pl.kernel,
    out_type=(
        jax.ShapeDtypeStruct((NPAD, FH), jnp.float32),
        jax.ShapeDtypeStruct((NPAD, FH), jnp.float32),
    ),
    mesh=_mesh,
    scratch_types=[
        pltpu.VMEM_SHARED((NPAD, FH), jnp.float32),
        pltpu.VMEM((IBCH, CHUNK), jnp.int32),
        pltpu.VMEM((IBCH, CHUNK), jnp.int32),
        pltpu.VMEM((CHUNK, FH), jnp.float32),
        pltpu.VMEM((CHUNK, FH), jnp.float32),
        pltpu.SemaphoreType.DMA,
    ],
)
def _seg1_sc(hwa, hwb, srcm, dstm, zeros2_hbm, outa, outb, agg, idxs, idxd, rows0, rows1, sem):
    c = lax.axis_index("c")
    s = lax.axis_index("s")
    pltpu.sync_copy(
        zeros2_hbm.at[pl.ds(s * RPT, RPT)], agg.at[pl.ds(s * RPT, RPT)]
    )
    plsc.subcore_barrier()

    def run(tab):
        _seg_loop(tab, agg, srcm, dstm, s * S1CH, S1CH, idxs, idxd, rows0, rows1, sem)

    @pl.when(c == 0)
    def _():
        run(hwa)

    @pl.when(c == 1)
    def _():
        run(hwb)

    plsc.subcore_barrier()

    def wout(o):
        pltpu.sync_copy(agg.at[pl.ds(s * RPT, RPT)], o.at[pl.ds(s * RPT, RPT)])

    @pl.when(c == 0)
    def _():
        wout(outa)

    @pl.when(c == 1)
    def _():
        wout(outb)


S2CH = ECH // (NC * NS)


@functools.partial(
    pl.kernel,
    out_type=(
        jax.ShapeDtypeStruct((NPAD, 2 * ZD), jnp.float32),
        jax.ShapeDtypeStruct((NPAD, 2 * ZD), jnp.float32),
    ),
    mesh=_mesh,
    scratch_types=[
        pltpu.VMEM_SHARED((NPAD, FH), jnp.float32),
        pltpu.VMEM((IBCH, CHUNK), jnp.int32),
        pltpu.VMEM((IBCH, CHUNK), jnp.int32),
        pltpu.VMEM((CHUNK, FH), jnp.float32),
        pltpu.VMEM((CHUNK, FH), jnp.float32),
        pltpu.SemaphoreType.DMA,
    ],
)
def _seg2_sc(tab, srcm, dstm, zeros2_hbm, out0, out1, agg, idxs, idxd, rows0, rows1, sem):
    c = lax.axis_index("c")
    s = lax.axis_index("s")
    pltpu.sync_copy(
        zeros2_hbm.at[pl.ds(s * RPT, RPT)], agg.at[pl.ds(s * RPT, RPT)]
    )
    plsc.subcore_barrier()
    base = (c * NS + s) * S2CH
    _seg_loop(tab, agg, srcm, dstm, base, S2CH, idxs, idxd, rows0, rows1, sem)
    plsc.subcore_barrier()

    def wout(o):
        pltpu.sync_copy(agg.at[pl.ds(s * RPT, RPT)], o.at[pl.ds(s * RPT, RPT)])

    @pl.when(c == 0)
    def _():
        wout(out0)

    @pl.when(c == 1)
    def _():
        wout(out1)


RB1 = 1024


def _mm1_body(x_ref, w_ref, deg_ref, outa_ref, outb_ref):
    ns = lax.rsqrt(jnp.maximum(deg_ref[...], 1.0))
    y = jnp.dot(x_ref[...], w_ref[...], preferred_element_type=jnp.float32)
    y = y * ns[:, None]
    outa_ref[...] = y[:, :FH]
    outb_ref[...] = y[:, FH:]


_mm1 = pl.pallas_call(
    _mm1_body,
    grid=(NPAD // RB1,),
    in_specs=[
        pl.BlockSpec((RB1, DD), lambda i: (i, 0)),
        pl.BlockSpec((DD, HH), lambda i: (0, 0)),
        pl.BlockSpec((RB1,), lambda i: (i,)),
    ],
    out_specs=[
        pl.BlockSpec((RB1, FH), lambda i: (i, 0)),
        pl.BlockSpec((RB1, FH), lambda i: (i, 0)),
    ],
    out_shape=[
        jax.ShapeDtypeStruct((NPAD, FH), jnp.float32),
        jax.ShapeDtypeStruct((NPAD, FH), jnp.float32),
    ],
)


def _mm2_body(a_ref, b_ref, degd_ref, degs_ref, w_ref, out_ref):
    nd = lax.rsqrt(jnp.maximum(degd_ref[...], 1.0))
    ns = lax.rsqrt(jnp.maximum(degs_ref[...], 1.0))
    h = jnp.concatenate([a_ref[...], b_ref[...]], axis=1) * nd[:, None]
    h = jnp.maximum(h, 0.0)
    y = jnp.dot(h, w_ref[...], preferred_element_type=jnp.float32)
    out_ref[...] = y * ns[:, None]


_mm2 = pl.pallas_call(
    _mm2_body,
    grid=(NPAD // RB1,),
    in_specs=[
        pl.BlockSpec((RB1, FH), lambda i: (i, 0)),
        pl.BlockSpec((RB1, FH), lambda i: (i, 0)),
        pl.BlockSpec((RB1,), lambda i: (i,)),
        pl.BlockSpec((RB1,), lambda i: (i,)),
        pl.BlockSpec((HH, 2 * ZD), lambda i: (0, 0)),
    ],
    out_specs=pl.BlockSpec((RB1, 2 * ZD), lambda i: (i, 0)),
    out_shape=jax.ShapeDtypeStruct((NPAD, 2 * ZD), jnp.float32),
)


BR = 512
BC = 1024
GI = -(-NN // BR)
GJ = -(-NN // BC)


def _dec_body(p0i, p1i, di, ei, p0j, p1j, dj, ej, out_ref):
    def mkz(p0, p1, dg, ep):
        nd = lax.rsqrt(jnp.maximum(dg[...], 1.0))[:, None]
        sm = p0[...] + p1[...]
        mu = sm[:, :ZD] * nd
        log_var = sm[:, ZD:] * nd
        return mu + jnp.exp(0.5 * log_var) * ep[...]

    zi = mkz(p0i, p1i, di, ei)
    zj = mkz(p0j, p1j, dj, ej)
    prod = lax.dot_general(
        zi, zj, (((1,), (1,)), ((), ())), preferred_element_type=jnp.float32
    )
    out_ref[...] = jax.nn.sigmoid(prod)


_dec = pl.pallas_call(
    _dec_body,
    grid=(GI, GJ),
    in_specs=[
        pl.BlockSpec((BR, 2 * ZD), lambda i, j: (i, 0)),
        pl.BlockSpec((BR, 2 * ZD), lambda i, j: (i, 0)),
        pl.BlockSpec((BR,), lambda i, j: (i,)),
        pl.BlockSpec((BR, ZD), lambda i, j: (i, 0)),
        pl.BlockSpec((BC, 2 * ZD), lambda i, j: (j, 0)),
        pl.BlockSpec((BC, 2 * ZD), lambda i, j: (j, 0)),
        pl.BlockSpec((BC,), lambda i, j: (j,)),
        pl.BlockSpec((BC, ZD), lambda i, j: (j, 0)),
    ],
    out_specs=pl.BlockSpec((BR, BC), lambda i, j: (i, j)),
    out_shape=jax.ShapeDtypeStruct((NN, NN), jnp.float32),
)


def kernel(x, edge_index, W1, W_mu, W_logvar, eps):
    src = edge_index[0]
    dst = edge_index[1]
    pad = jnp.full((EPAD - EE,), DUMMY, jnp.int32)
    srcm = jnp.concatenate([src, pad]).reshape(ECH, CHUNK)
    dstm = jnp.concatenate([dst, pad]).reshape(ECH, CHUNK)
    x_pad = jnp.pad(x, ((0, NPAD - NN), (0, 0)))
    eps_pad = jnp.pad(eps, ((0, NPAD - NN), (0, 0)))
    zeros1 = jnp.zeros((NPAD,), jnp.float32)
    zeros2 = jnp.zeros((NPAD, FH), jnp.float32)
    ones1 = jnp.ones((CHUNK,), jnp.float32)
    w_cat = jnp.concatenate([W_mu, W_logvar], axis=1)

    deg = _deg_sc(srcm, dstm, zeros1, ones1)
    deg_src = deg[0]
    deg_dst = deg[1]
    hwa, hwb = _mm1(x_pad, W1, deg_src)
    agga, aggb = _seg1_sc(hwa, hwb, srcm, dstm, zeros2)
    mm2o = _mm2(agga, aggb, deg_dst, deg_src, w_cat)
    p0, p1 = _seg2_sc(mm2o, srcm, dstm, zeros2)
    return _dec(p0, p1, deg_dst, eps_pad, p0, p1, deg_dst, eps_pad)
